# baseline (device time: 50379 ns/iter reference)
import jax
import jax.numpy as jnp
from jax import lax
from jax.experimental import pallas as pl
from jax.experimental.pallas import tpu as pltpu

N_DEV = 8
S = 2


def kernel(x, w_mat):
    m_full, k_per = x.shape
    k_per2, n = w_mat.shape
    assert k_per == k_per2
    m_per = m_full // N_DEV
    stripe = n // (2 * S)

    def body(x_ref, w_ref, out_ref,
             comm_cw, comm_ccw, xf_ref, wf_ref,
             send_cw, recv_cw, send_ccw, recv_ccw):
        my = lax.axis_index("i")

        def ring2log(q):
            return jnp.where(q < 4, q, 11 - q)

        p = ring2log(my)
        left = ring2log(lax.rem(p + (N_DEV - 1), N_DEV))
        right = ring2log(lax.rem(p + 1, N_DEV))

        barrier_sem = pltpu.get_barrier_semaphore()
        for nbr in (left, right):
            pl.semaphore_signal(
                barrier_sem, inc=1,
                device_id=(nbr,), device_id_type=pl.DeviceIdType.MESH,
            )
        pl.semaphore_wait(barrier_sem, 2)

        xf_ref[...] = x_ref[...].astype(jnp.float32)
        wf_ref[...] = w_ref[...].astype(jnp.float32)

        def stripe_dot(c, st):
            xb = xf_ref[pl.ds(c * m_per, m_per), :]
            wb = wf_ref[:, st * stripe:(st + 1) * stripe]
            return jnp.dot(xb, wb, preferred_element_type=jnp.float32)

        def c_cw(h):
            return ring2log(lax.rem(p + (2 * N_DEV - 2 - h), N_DEV))

        def c_ccw(h):
            return ring2log(lax.rem(p + 2 + h, N_DEV))

        def mk(h, s, cw):
            comm = comm_cw if cw else comm_ccw
            return pltpu.make_async_remote_copy(
                src_ref=comm.at[s, h],
                dst_ref=comm.at[s, h + 1],
                send_sem=(send_cw if cw else send_ccw).at[s, h],
                recv_sem=(recv_cw if cw else recv_ccw).at[s, h],
                device_id=(right if cw else left,),
                device_id_type=pl.DeviceIdType.MESH,
            )

        descrs = {}
        for s in range(S):
            comm_cw[s, 0] = stripe_dot(left, s).astype(jnp.bfloat16)
            d = mk(0, s, True)
            d.start()
            descrs[(0, s, True)] = d
            comm_ccw[s, 0] = stripe_dot(right, S + s).astype(jnp.bfloat16)
            d = mk(0, s, False)
            d.start()
            descrs[(0, s, False)] = d

        for h in range(N_DEV - 1):
            for s in range(S):
                for cw in (True, False):
                    st = s if cw else S + s
                    c = c_cw(h) if cw else c_ccw(h)
                    last = h == N_DEV - 2
                    bl = stripe_dot(c, st)
                    d = descrs[(h, s, cw)]
                    d.wait_recv()
                    comm = comm_cw if cw else comm_ccw
                    if not last:
                        comm[s, h + 1] = (
                            comm[s, h + 1] + bl.astype(jnp.bfloat16))
                        d2 = mk(h + 1, s, cw)
                        d2.start()
                        descrs[(h + 1, s, cw)] = d2
                    else:
                        acc = comm[s, h + 1].astype(jnp.float32) + bl
                        out_ref[:, st * stripe:(st + 1) * stripe] = (
                            jnp.maximum(acc, 0.0))

        for d in descrs.values():
            d.wait_send()

    return pl.pallas_call(
        body,
        out_shape=jax.ShapeDtypeStruct((m_per, n), jnp.float32),
        in_specs=[
            pl.BlockSpec(memory_space=pltpu.VMEM),
            pl.BlockSpec(memory_space=pltpu.VMEM),
        ],
        out_specs=pl.BlockSpec(memory_space=pltpu.VMEM),
        scratch_shapes=[
            pltpu.VMEM((S, N_DEV, m_per, stripe), jnp.bfloat16),
            pltpu.VMEM((S, N_DEV, m_per, stripe), jnp.bfloat16),
            pltpu.VMEM((m_full, k_per), jnp.float32),
            pltpu.VMEM((k_per, n), jnp.float32),
            pltpu.SemaphoreType.DMA((S, N_DEV - 1)),
            pltpu.SemaphoreType.DMA((S, N_DEV - 1)),
            pltpu.SemaphoreType.DMA((S, N_DEV - 1)),
            pltpu.SemaphoreType.DMA((S, N_DEV - 1)),
        ],
        compiler_params=pltpu.CompilerParams(collective_id=0),
    )(x, w_mat)


# device time: 43605 ns/iter; 1.1553x vs baseline; 1.1553x over previous
import jax
import jax.numpy as jnp
from jax import lax
from jax.experimental import pallas as pl
from jax.experimental.pallas import tpu as pltpu

N_DEV = 8

MASK_X, MASK_Y, MASK_Z = 1, 3, 4

PARTS = (
    (0, 768, (MASK_X, MASK_Y, MASK_Z)),
    (768, 640, (MASK_Y, MASK_Z, MASK_X)),
    (1408, 640, (MASK_Z, MASK_X, MASK_Y)),
)


def kernel(x, w_mat):
    m_full, k_per = x.shape
    k_per2, n = w_mat.shape
    assert k_per == k_per2
    m_per = m_full // N_DEV
    assert n == sum(p[1] for p in PARTS)

    def body(x_ref, w_ref, out_ref,
             acc_a, acc_b, acc_c, rcv_a, rcv_b, rcv_c,
             ssem_a, ssem_b, ssem_c, rsem_a, rsem_b, rsem_c):
        my = lax.axis_index("i")
        accs = (acc_a, acc_b, acc_c)
        rcvs = (rcv_a, rcv_b, rcv_c)
        ssems = (ssem_a, ssem_b, ssem_c)
        rsems = (rsem_a, rsem_b, rsem_c)

        barrier_sem = pltpu.get_barrier_semaphore()
        for mask in (MASK_X, MASK_Y, MASK_Z):
            pl.semaphore_signal(
                barrier_sem, inc=1,
                device_id=(my ^ mask,), device_id_type=pl.DeviceIdType.MESH,
            )
        pl.semaphore_wait(barrier_sem, 3)

        def chunk_of_slot(pi, k):
            d1, d2, d3 = PARTS[pi][2]
            m = (d1 if k & 4 else 0) ^ (d2 if k & 2 else 0) ^ (d3 if k & 1 else 0)
            return my ^ m

        def slot_dot(pi, k):
            off, w, _ = PARTS[pi]
            c = chunk_of_slot(pi, k)
            xb = x_ref[pl.ds(c * m_per, m_per), :]
            wb = w_ref[:, off:off + w]
            return jnp.dot(
                xb, wb, preferred_element_type=jnp.float32
            ).astype(jnp.bfloat16)

        RND = ((4, 0), (2, 4), (1, 6))

        def mk(pi, r):
            half, rlo = RND[r]
            return pltpu.make_async_remote_copy(
                src_ref=accs[pi].at[pl.ds(half, half)],
                dst_ref=rcvs[pi].at[pl.ds(rlo, half)],
                send_sem=ssems[pi].at[r],
                recv_sem=rsems[pi].at[r],
                device_id=(my ^ PARTS[pi][2][r],),
                device_id_type=pl.DeviceIdType.MESH,
            )

        descrs = {}

        for pi in range(3):
            for k in range(4, 8):
                accs[pi][k] = slot_dot(pi, k)
            d = mk(pi, 0)
            d.start()
            descrs[(pi, 0)] = d
        for pi in range(3):
            for k in range(4):
                accs[pi][k] = slot_dot(pi, k)

        part_order = (1, 2, 0)

        for pi in part_order:
            descrs[(pi, 0)].wait_recv()
            accs[pi][2] = accs[pi][2] + rcv_slot(rcvs[pi], 2)
            accs[pi][3] = accs[pi][3] + rcv_slot(rcvs[pi], 3)
            d = mk(pi, 1)
            d.start()
            descrs[(pi, 1)] = d
        for pi in part_order:
            accs[pi][0] = accs[pi][0] + rcv_slot(rcvs[pi], 0)
            accs[pi][1] = accs[pi][1] + rcv_slot(rcvs[pi], 1)

        for pi in part_order:
            descrs[(pi, 1)].wait_recv()
            accs[pi][1] = accs[pi][1] + rcv_slot(rcvs[pi], 5)
            d = mk(pi, 2)
            d.start()
            descrs[(pi, 2)] = d
        for pi in part_order:
            accs[pi][0] = accs[pi][0] + rcv_slot(rcvs[pi], 4)

        for pi in part_order:
            off, w, _ = PARTS[pi]
            descrs[(pi, 2)].wait_recv()
            acc = (accs[pi][0].astype(jnp.float32)
                   + rcv_slot(rcvs[pi], 6).astype(jnp.float32))
            out_ref[:, off:off + w] = jnp.maximum(acc, 0.0)

        for d in descrs.values():
            d.wait_send()

    def rcv_slot(rcv, k):
        return rcv[k]

    scratch = []
    for _, w, _ in PARTS:
        scratch.append(pltpu.VMEM((N_DEV, m_per, w), jnp.bfloat16))
    for _, w, _ in PARTS:
        scratch.append(pltpu.VMEM((N_DEV - 1, m_per, w), jnp.bfloat16))
    scratch += [pltpu.SemaphoreType.DMA((3,))] * 6

    return pl.pallas_call(
        body,
        out_shape=jax.ShapeDtypeStruct((m_per, n), jnp.float32),
        in_specs=[
            pl.BlockSpec(memory_space=pltpu.VMEM),
            pl.BlockSpec(memory_space=pltpu.VMEM),
        ],
        out_specs=pl.BlockSpec(memory_space=pltpu.VMEM),
        scratch_shapes=scratch,
        compiler_params=pltpu.CompilerParams(collective_id=0),
    )(x, w_mat)


# device time: 38667 ns/iter; 1.3029x vs baseline; 1.1277x over previous
import jax
import jax.numpy as jnp
from jax import lax
from jax.experimental import pallas as pl
from jax.experimental.pallas import tpu as pltpu

N_DEV = 8

MASK_X, MASK_Y, MASK_Z = 1, 3, 4

PARTS = (
    (0, 384, (MASK_X, MASK_Y, MASK_Z)),
    (384, 384, (MASK_X, MASK_Y, MASK_Z)),
    (768, 384, (MASK_Y, MASK_Z, MASK_X)),
    (1152, 256, (MASK_Y, MASK_Z, MASK_X)),
    (1408, 384, (MASK_Z, MASK_X, MASK_Y)),
    (1792, 256, (MASK_Z, MASK_X, MASK_Y)),
)
NP = len(PARTS)
PART_ORDER = (0, 2, 4, 3, 5, 1)


def kernel(x, w_mat):
    m_full, k_per = x.shape
    k_per2, n = w_mat.shape
    assert k_per == k_per2
    m_per = m_full // N_DEV
    assert n == sum(p[1] for p in PARTS)

    def body(x_ref, w_ref, out_ref, *scratch):
        my = lax.axis_index("i")
        accs = scratch[:NP]
        rcvs = scratch[NP:2 * NP]
        ssems = scratch[2 * NP:3 * NP]
        rsems = scratch[3 * NP:4 * NP]

        barrier_sem = pltpu.get_barrier_semaphore()
        for mask in (MASK_X, MASK_Y, MASK_Z):
            pl.semaphore_signal(
                barrier_sem, inc=1,
                device_id=(my ^ mask,), device_id_type=pl.DeviceIdType.MESH,
            )
        pl.semaphore_wait(barrier_sem, 3)

        def chunk_of_slot(pi, k):
            d1, d2, d3 = PARTS[pi][2]
            m = (d1 if k & 4 else 0) ^ (d2 if k & 2 else 0) ^ (d3 if k & 1 else 0)
            return my ^ m

        def slot_dot(pi, k):
            off, w, _ = PARTS[pi]
            c = chunk_of_slot(pi, k)
            xb = x_ref[pl.ds(c * m_per, m_per), :]
            wb = w_ref[:, off:off + w]
            return jnp.dot(
                xb, wb, preferred_element_type=jnp.float32
            ).astype(jnp.bfloat16)

        RND = ((4, 0), (2, 4), (1, 6))

        def mk(pi, r):
            half, rlo = RND[r]
            return pltpu.make_async_remote_copy(
                src_ref=accs[pi].at[pl.ds(half, half)],
                dst_ref=rcvs[pi].at[pl.ds(rlo, half)],
                send_sem=ssems[pi].at[r],
                recv_sem=rsems[pi].at[r],
                device_id=(my ^ PARTS[pi][2][r],),
                device_id_type=pl.DeviceIdType.MESH,
            )

        descrs = {}

        for pi in range(NP):
            for k in range(4, 8):
                accs[pi][k] = slot_dot(pi, k)
            d = mk(pi, 0)
            d.start()
            descrs[(pi, 0)] = d
        for pi in range(NP):
            for k in range(4):
                accs[pi][k] = slot_dot(pi, k)

        part_order = PART_ORDER

        for pi in part_order:
            descrs[(pi, 0)].wait_recv()
            accs[pi][2] = accs[pi][2] + rcv_slot(rcvs[pi], 2)
            accs[pi][3] = accs[pi][3] + rcv_slot(rcvs[pi], 3)
            d = mk(pi, 1)
            d.start()
            descrs[(pi, 1)] = d
        for pi in part_order:
            accs[pi][0] = accs[pi][0] + rcv_slot(rcvs[pi], 0)
            accs[pi][1] = accs[pi][1] + rcv_slot(rcvs[pi], 1)

        for pi in part_order:
            descrs[(pi, 1)].wait_recv()
            accs[pi][1] = accs[pi][1] + rcv_slot(rcvs[pi], 5)
            d = mk(pi, 2)
            d.start()
            descrs[(pi, 2)] = d
        for pi in part_order:
            accs[pi][0] = accs[pi][0] + rcv_slot(rcvs[pi], 4)

        for pi in part_order:
            off, w, _ = PARTS[pi]
            descrs[(pi, 2)].wait_recv()
            acc = (accs[pi][0].astype(jnp.float32)
                   + rcv_slot(rcvs[pi], 6).astype(jnp.float32))
            out_ref[:, off:off + w] = jnp.maximum(acc, 0.0)

        for d in descrs.values():
            d.wait_send()

    def rcv_slot(rcv, k):
        return rcv[k]

    scratch = []
    for _, w, _ in PARTS:
        scratch.append(pltpu.VMEM((N_DEV, m_per, w), jnp.bfloat16))
    for _, w, _ in PARTS:
        scratch.append(pltpu.VMEM((N_DEV - 1, m_per, w), jnp.bfloat16))
    scratch += [pltpu.SemaphoreType.DMA((3,))] * (2 * NP)

    return pl.pallas_call(
        body,
        out_shape=jax.ShapeDtypeStruct((m_per, n), jnp.float32),
        in_specs=[
            pl.BlockSpec(memory_space=pltpu.VMEM),
            pl.BlockSpec(memory_space=pltpu.VMEM),
        ],
        out_specs=pl.BlockSpec(memory_space=pltpu.VMEM),
        scratch_shapes=scratch,
        compiler_params=pltpu.CompilerParams(collective_id=0),
    )(x, w_mat)


# device time: 38083 ns/iter; 1.3229x vs baseline; 1.0153x over previous
import jax
import jax.numpy as jnp
from jax import lax
from jax.experimental import pallas as pl
from jax.experimental.pallas import tpu as pltpu

N_DEV = 8

MASK_X, MASK_Y, MASK_Z = 1, 3, 4

PARTS = (
    (0, 256, (MASK_X, MASK_Y, MASK_Z)),
    (256, 256, (MASK_X, MASK_Y, MASK_Z)),
    (512, 256, (MASK_X, MASK_Y, MASK_Z)),
    (768, 256, (MASK_Y, MASK_Z, MASK_X)),
    (1024, 256, (MASK_Y, MASK_Z, MASK_X)),
    (1280, 128, (MASK_Y, MASK_Z, MASK_X)),
    (1408, 256, (MASK_Z, MASK_X, MASK_Y)),
    (1664, 256, (MASK_Z, MASK_X, MASK_Y)),
    (1920, 128, (MASK_Z, MASK_X, MASK_Y)),
)
NP = len(PARTS)
PART_ORDER = (0, 3, 6, 1, 4, 7, 5, 8, 2)


def kernel(x, w_mat):
    m_full, k_per = x.shape
    k_per2, n = w_mat.shape
    assert k_per == k_per2
    m_per = m_full // N_DEV
    assert n == sum(p[1] for p in PARTS)

    def body(x_ref, w_ref, out_ref, *scratch):
        my = lax.axis_index("i")
        accs = scratch[:NP]
        rcvs = scratch[NP:2 * NP]
        ssems = scratch[2 * NP:3 * NP]
        rsems = scratch[3 * NP:4 * NP]

        barrier_sem = pltpu.get_barrier_semaphore()
        for mask in (MASK_X, MASK_Y, MASK_Z):
            pl.semaphore_signal(
                barrier_sem, inc=1,
                device_id=(my ^ mask,), device_id_type=pl.DeviceIdType.MESH,
            )
        pl.semaphore_wait(barrier_sem, 3)

        def chunk_of_slot(pi, k):
            d1, d2, d3 = PARTS[pi][2]
            m = (d1 if k & 4 else 0) ^ (d2 if k & 2 else 0) ^ (d3 if k & 1 else 0)
            return my ^ m

        def slot_dot(pi, k):
            off, w, _ = PARTS[pi]
            c = chunk_of_slot(pi, k)
            xb = x_ref[pl.ds(c * m_per, m_per), :]
            wb = w_ref[:, off:off + w]
            return jnp.dot(
                xb, wb, preferred_element_type=jnp.float32
            ).astype(jnp.bfloat16)

        RND = ((4, 0), (2, 4), (1, 6))

        def mk(pi, r):
            half, rlo = RND[r]
            return pltpu.make_async_remote_copy(
                src_ref=accs[pi].at[pl.ds(half, half)],
                dst_ref=rcvs[pi].at[pl.ds(rlo, half)],
                send_sem=ssems[pi].at[r],
                recv_sem=rsems[pi].at[r],
                device_id=(my ^ PARTS[pi][2][r],),
                device_id_type=pl.DeviceIdType.MESH,
            )

        descrs = {}

        for pi in range(NP):
            for k in range(4, 8):
                accs[pi][k] = slot_dot(pi, k)
            d = mk(pi, 0)
            d.start()
            descrs[(pi, 0)] = d
        for pi in range(NP):
            for k in range(4):
                accs[pi][k] = slot_dot(pi, k)

        part_order = PART_ORDER

        for pi in part_order:
            descrs[(pi, 0)].wait_recv()
            accs[pi][2] = accs[pi][2] + rcv_slot(rcvs[pi], 2)
            accs[pi][3] = accs[pi][3] + rcv_slot(rcvs[pi], 3)
            d = mk(pi, 1)
            d.start()
            descrs[(pi, 1)] = d
        for pi in part_order:
            accs[pi][0] = accs[pi][0] + rcv_slot(rcvs[pi], 0)
            accs[pi][1] = accs[pi][1] + rcv_slot(rcvs[pi], 1)

        for pi in part_order:
            descrs[(pi, 1)].wait_recv()
            accs[pi][1] = accs[pi][1] + rcv_slot(rcvs[pi], 5)
            d = mk(pi, 2)
            d.start()
            descrs[(pi, 2)] = d
        for pi in part_order:
            accs[pi][0] = accs[pi][0] + rcv_slot(rcvs[pi], 4)

        for pi in part_order:
            off, w, _ = PARTS[pi]
            descrs[(pi, 2)].wait_recv()
            acc = (accs[pi][0].astype(jnp.float32)
                   + rcv_slot(rcvs[pi], 6).astype(jnp.float32))
            out_ref[:, off:off + w] = jnp.maximum(acc, 0.0)

        for d in descrs.values():
            d.wait_send()

    def rcv_slot(rcv, k):
        return rcv[k]

    scratch = []
    for _, w, _ in PARTS:
        scratch.append(pltpu.VMEM((N_DEV, m_per, w), jnp.bfloat16))
    for _, w, _ in PARTS:
        scratch.append(pltpu.VMEM((N_DEV - 1, m_per, w), jnp.bfloat16))
    scratch += [pltpu.SemaphoreType.DMA((3,))] * (2 * NP)

    return pl.pallas_call(
        body,
        out_shape=jax.ShapeDtypeStruct((m_per, n), jnp.float32),
        in_specs=[
            pl.BlockSpec(memory_space=pltpu.VMEM),
            pl.BlockSpec(memory_space=pltpu.VMEM),
        ],
        out_specs=pl.BlockSpec(memory_space=pltpu.VMEM),
        scratch_shapes=scratch,
        compiler_params=pltpu.CompilerParams(collective_id=0),
    )(x, w_mat)


# device time: 38002 ns/iter; 1.3257x vs baseline; 1.0021x over previous
import jax
import jax.numpy as jnp
from jax import lax
from jax.experimental import pallas as pl
from jax.experimental.pallas import tpu as pltpu

N_DEV = 8

MASK_X, MASK_Y, MASK_Z = 1, 3, 4

PARTS = (
    (0, 256, (MASK_X, MASK_Y, MASK_Z)),
    (256, 256, (MASK_X, MASK_Y, MASK_Z)),
    (512, 256, (MASK_X, MASK_Y, MASK_Z)),
    (768, 256, (MASK_Y, MASK_Z, MASK_X)),
    (1024, 256, (MASK_Y, MASK_Z, MASK_X)),
    (1280, 128, (MASK_Y, MASK_Z, MASK_X)),
    (1408, 256, (MASK_Z, MASK_X, MASK_Y)),
    (1664, 256, (MASK_Z, MASK_X, MASK_Y)),
    (1920, 128, (MASK_Z, MASK_X, MASK_Y)),
)
NP = len(PARTS)
PART_ORDER = (0, 3, 6, 1, 4, 7, 5, 8, 2)


def kernel(x, w_mat):
    m_full, k_per = x.shape
    k_per2, n = w_mat.shape
    assert k_per == k_per2
    m_per = m_full // N_DEV
    assert n == sum(p[1] for p in PARTS)

    def body(x_ref, w_ref, out_ref, *scratch):
        my = lax.axis_index("i")
        accs = scratch[:NP]
        rcvs = scratch[NP:2 * NP]
        ssems = scratch[2 * NP:3 * NP]
        rsems = scratch[3 * NP:4 * NP]

        barrier_sem = pltpu.get_barrier_semaphore()
        for mask in (MASK_X, MASK_Y, MASK_Z):
            pl.semaphore_signal(
                barrier_sem, inc=1,
                device_id=(my ^ mask,), device_id_type=pl.DeviceIdType.MESH,
            )
        pl.semaphore_wait(barrier_sem, 3)

        def chunk_of_slot(pi, k):
            d1, d2, d3 = PARTS[pi][2]
            m = (d1 if k & 4 else 0) ^ (d2 if k & 2 else 0) ^ (d3 if k & 1 else 0)
            return my ^ m

        def slot_dot(pi, k):
            off, w, _ = PARTS[pi]
            c = chunk_of_slot(pi, k)
            xb = x_ref[pl.ds(c * m_per, m_per), :]
            wb = w_ref[:, off:off + w]
            return jnp.dot(
                xb, wb, preferred_element_type=jnp.float32
            ).astype(jnp.bfloat16)

        RND = ((4, 0), (2, 4), (1, 6))

        def mk(pi, r):
            half, rlo = RND[r]
            return pltpu.make_async_remote_copy(
                src_ref=accs[pi].at[pl.ds(half, half)],
                dst_ref=rcvs[pi].at[pl.ds(rlo, half)],
                send_sem=ssems[pi].at[r],
                recv_sem=rsems[pi].at[r],
                device_id=(my ^ PARTS[pi][2][r],),
                device_id_type=pl.DeviceIdType.MESH,
            )

        descrs = {}

        for pi in (0, 3, 6, 1, 4, 7, 2, 5, 8):
            for k in range(4, 8):
                accs[pi][k] = slot_dot(pi, k)
            d = mk(pi, 0)
            d.start()
            descrs[(pi, 0)] = d
        for pi in range(NP):
            for k in range(4):
                accs[pi][k] = slot_dot(pi, k)

        part_order = PART_ORDER

        for pi in part_order:
            descrs[(pi, 0)].wait_recv()
            accs[pi][2] = accs[pi][2] + rcv_slot(rcvs[pi], 2)
            accs[pi][3] = accs[pi][3] + rcv_slot(rcvs[pi], 3)
            d = mk(pi, 1)
            d.start()
            descrs[(pi, 1)] = d
        for pi in part_order:
            accs[pi][0] = accs[pi][0] + rcv_slot(rcvs[pi], 0)
            accs[pi][1] = accs[pi][1] + rcv_slot(rcvs[pi], 1)

        for pi in part_order:
            descrs[(pi, 1)].wait_recv()
            accs[pi][1] = accs[pi][1] + rcv_slot(rcvs[pi], 5)
            d = mk(pi, 2)
            d.start()
            descrs[(pi, 2)] = d
        for pi in part_order:
            accs[pi][0] = accs[pi][0] + rcv_slot(rcvs[pi], 4)

        for pi in part_order:
            off, w, _ = PARTS[pi]
            descrs[(pi, 2)].wait_recv()
            acc = (accs[pi][0].astype(jnp.float32)
                   + rcv_slot(rcvs[pi], 6).astype(jnp.float32))
            out_ref[:, off:off + w] = jnp.maximum(acc, 0.0)

        for d in descrs.values():
            d.wait_send()

    def rcv_slot(rcv, k):
        return rcv[k]

    scratch = []
    for _, w, _ in PARTS:
        scratch.append(pltpu.VMEM((N_DEV, m_per, w), jnp.bfloat16))
    for _, w, _ in PARTS:
        scratch.append(pltpu.VMEM((N_DEV - 1, m_per, w), jnp.bfloat16))
    scratch += [pltpu.SemaphoreType.DMA((3,))] * (2 * NP)

    return pl.pallas_call(
        body,
        out_shape=jax.ShapeDtypeStruct((m_per, n), jnp.float32),
        in_specs=[
            pl.BlockSpec(memory_space=pltpu.VMEM),
            pl.BlockSpec(memory_space=pltpu.VMEM),
        ],
        out_specs=pl.BlockSpec(memory_space=pltpu.VMEM),
        scratch_shapes=scratch,
        compiler_params=pltpu.CompilerParams(collective_id=0),
    )(x, w_mat)
